# baseline (speedup 1.0000x reference)
import functools

import jax
import jax.numpy as jnp
from jax import lax
from jax.experimental import pallas as pl
from jax.experimental.pallas import tpu as pltpu
from jax.experimental.pallas import tpu_sc as plsc

K = 16


def _gmf_sc(B, NC, NS):
    NW = NC * NS
    b_per_w = B // NW
    n_blocks = b_per_w // K
    mesh = plsc.VectorSubcoreMesh(core_axis_name="c", subcore_axis_name="s")

    @functools.partial(
        pl.kernel,
        mesh=mesh,
        out_type=jax.ShapeDtypeStruct((B,), jnp.float32),
        compiler_params=pltpu.CompilerParams(
            needs_layout_passes=False, use_tc_tiling_on_sc=True),
        scratch_types=[
            pltpu.VMEM((b_per_w,), jnp.int32),
            pltpu.VMEM((b_per_w,), jnp.int32),
            pltpu.VMEM((b_per_w // 2, K), jnp.float32),
            pltpu.VMEM((b_per_w // 2, K), jnp.float32),
            pltpu.VMEM((K,), jnp.float32),
            pltpu.VMEM((K,), jnp.float32),
            pltpu.VMEM((b_per_w,), jnp.float32),
            pltpu.SemaphoreType.DMA,
        ],
    )
    def gmf(user_hbm, item_hbm, ut_hbm, it_hbm, w_hbm, b_hbm, out_hbm,
            uidxv, iidxv, ubuf, ibuf, wv, bv, outv, sem):
        wid = lax.axis_index("s") * NC + lax.axis_index("c")
        base = wid * b_per_w

        pltpu.sync_copy(w_hbm, wv)
        pltpu.sync_copy(b_hbm, bv)
        pltpu.sync_copy(user_hbm.at[pl.ds(base, b_per_w)], uidxv)
        pltpu.sync_copy(item_hbm.at[pl.ds(base, b_per_w)], iidxv)

        wvec = wv[...]
        bias = bv[...]
        lane = lax.iota(jnp.int32, K)
        masks = [lane == j for j in range(K)]
        izero = jnp.zeros((K,), jnp.int32)

        half_rows = b_per_w // 2
        half_blocks = n_blocks // 2

        for half in range(2):
            hoff = half * half_blocks * K

            def issue_body(blk, carry):
                uv = uidxv[pl.ds(hoff + blk * K, K)]
                iv = iidxv[pl.ds(hoff + blk * K, K)]
                for j in range(K):
                    row = blk * K + j
                    ui = jnp.sum(jnp.where(masks[j], uv, izero))
                    ii = jnp.sum(jnp.where(masks[j], iv, izero))
                    pltpu.async_copy(ut_hbm.at[ui], ubuf.at[row], sem)
                    pltpu.async_copy(it_hbm.at[ii], ibuf.at[row], sem)
                return carry

            def compute_body(blk, carry):
                acc = jnp.zeros((K,), jnp.float32)
                for j in range(K):
                    row = blk * K + j
                    u = ubuf[row]
                    v = ibuf[row]
                    s = jnp.sum(u * v * wvec)
                    acc = jnp.where(masks[j], s, acc)
                outv[pl.ds(hoff + blk * K, K)] = (
                    1.0 / (1.0 + jnp.exp(-(acc + bias))))
                return carry

            lax.fori_loop(0, half_blocks, issue_body, 0)
            pltpu.make_async_copy(
                ut_hbm.at[pl.ds(0, half_rows)], ubuf, sem).wait()
            pltpu.make_async_copy(
                it_hbm.at[pl.ds(0, half_rows)], ibuf, sem).wait()
            lax.fori_loop(0, half_blocks, compute_body, 0)
        pltpu.sync_copy(outv, out_hbm.at[pl.ds(base, b_per_w)])

    return gmf


_TBLK = 512


def _transpose_tc(tblT):
    n = tblT.shape[1]

    def body(in_ref, out_ref):
        out_ref[...] = in_ref[...].T

    return pl.pallas_call(
        body,
        grid=(pl.cdiv(n, _TBLK),),
        in_specs=[pl.BlockSpec((K, _TBLK), lambda i: (0, i))],
        out_specs=pl.BlockSpec((_TBLK, K), lambda i: (i, 0)),
        out_shape=jax.ShapeDtypeStruct((n, K), jnp.float32),
    )(tblT)


def kernel(user, item, user_table, item_table, W, b):
    B = user.shape[0]
    info = plsc.get_sparse_core_info()
    NC, NS = info.num_cores, info.num_subcores

    user_i = user.astype(jnp.int32)
    item_i = item.astype(jnp.int32)
    w_vec = W.reshape(K)
    b_splat = jnp.broadcast_to(b.reshape(1), (K,))

    ut_rm = _transpose_tc(user_table.T)
    it_rm = _transpose_tc(item_table.T)

    out = _gmf_sc(B, NC, NS)(user_i, item_i, ut_rm, it_rm,
                             w_vec, b_splat)
    return out.reshape(B, 1)


# speedup vs baseline: 2.5700x; 2.5700x over previous
import functools

import jax
import jax.numpy as jnp
from jax import lax
from jax.experimental import pallas as pl
from jax.experimental.pallas import tpu as pltpu
from jax.experimental.pallas import tpu_sc as plsc

K = 16


def _gmf_sc(B, NC, NS):
    NW = NC * NS
    b_per_w = B // NW
    n_blocks = b_per_w // K
    mesh = plsc.VectorSubcoreMesh(core_axis_name="c", subcore_axis_name="s")

    @functools.partial(
        pl.kernel,
        mesh=mesh,
        out_type=jax.ShapeDtypeStruct((B,), jnp.float32),
        compiler_params=pltpu.CompilerParams(
            needs_layout_passes=False, use_tc_tiling_on_sc=True),
        scratch_types=[
            pltpu.VMEM((b_per_w,), jnp.int32),
            pltpu.VMEM((b_per_w,), jnp.int32),
            pltpu.VMEM((b_per_w // 2, K), jnp.float32),
            pltpu.VMEM((b_per_w // 2, K), jnp.float32),
            pltpu.VMEM((K,), jnp.float32),
            pltpu.VMEM((K,), jnp.float32),
            pltpu.VMEM((b_per_w,), jnp.float32),
            pltpu.SemaphoreType.DMA,
        ],
    )
    def gmf(user_hbm, item_hbm, ut_hbm, it_hbm, w_hbm, b_hbm, out_hbm,
            uidxv, iidxv, ubuf, ibuf, wv, bv, outv, sem):
        wid = lax.axis_index("s") * NC + lax.axis_index("c")
        base = wid * b_per_w

        pltpu.sync_copy(w_hbm, wv)
        pltpu.sync_copy(b_hbm, bv)
        pltpu.sync_copy(user_hbm.at[pl.ds(base, b_per_w)], uidxv)
        pltpu.sync_copy(item_hbm.at[pl.ds(base, b_per_w)], iidxv)

        wvec = wv[...]
        bias = bv[...]
        lane = lax.iota(jnp.int32, K)
        masks = [lane == j for j in range(K)]
        izero = jnp.zeros((K,), jnp.int32)

        half_rows = b_per_w // 2
        half_blocks = n_blocks // 2

        for half in range(2):
            hoff = half * half_blocks * K

            def issue_body(blk, carry):
                uv = uidxv[pl.ds(hoff + blk * K, K)]
                iv = iidxv[pl.ds(hoff + blk * K, K)]
                for j in range(K):
                    row = blk * K + j
                    ui = jnp.sum(jnp.where(masks[j], uv, izero))
                    ii = jnp.sum(jnp.where(masks[j], iv, izero))
                    pltpu.async_copy(ut_hbm.at[ui], ubuf.at[row], sem)
                    pltpu.async_copy(it_hbm.at[ii], ibuf.at[row], sem)
                return carry

            def compute_body(blk, carry):
                acc = jnp.zeros((K,), jnp.float32)
                for j in range(K):
                    row = blk * K + j
                    u = ubuf[row]
                    v = ibuf[row]
                    s = jnp.sum(u * v * wvec)
                    acc = jnp.where(masks[j], s, acc)
                outv[pl.ds(hoff + blk * K, K)] = (
                    1.0 / (1.0 + jnp.exp(-(acc + bias))))
                return carry

            lax.fori_loop(0, half_blocks, issue_body, 0)
            pltpu.make_async_copy(
                ut_hbm.at[pl.ds(0, half_rows)], ubuf, sem).wait()
            pltpu.make_async_copy(
                it_hbm.at[pl.ds(0, half_rows)], ibuf, sem).wait()
            lax.fori_loop(0, half_blocks, compute_body, 0)
        pltpu.sync_copy(outv, out_hbm.at[pl.ds(base, b_per_w)])

    return gmf


_TBLK = 2048


def _transpose_tc(tblT):
    n = tblT.shape[1]

    def body(in_ref, out_ref):
        x = in_ref[...]
        eye = (lax.broadcasted_iota(jnp.int32, (K, K), 0) ==
               lax.broadcasted_iota(jnp.int32, (K, K), 1)).astype(jnp.float32)
        out_ref[...] = lax.dot_general(
            x, eye, (((0,), (0,)), ((), ())),
            preferred_element_type=jnp.float32)

    return pl.pallas_call(
        body,
        grid=(pl.cdiv(n, _TBLK),),
        in_specs=[pl.BlockSpec((K, _TBLK), lambda i: (0, i))],
        out_specs=pl.BlockSpec((_TBLK, K), lambda i: (i, 0)),
        out_shape=jax.ShapeDtypeStruct((n, K), jnp.float32),
    )(tblT)


def kernel(user, item, user_table, item_table, W, b):
    B = user.shape[0]
    info = plsc.get_sparse_core_info()
    NC, NS = info.num_cores, info.num_subcores

    user_i = user.astype(jnp.int32)
    item_i = item.astype(jnp.int32)
    w_vec = W.reshape(K)
    b_splat = jnp.broadcast_to(b.reshape(1), (K,))

    ut_rm = _transpose_tc(user_table.T)
    it_rm = _transpose_tc(item_table.T)

    out = _gmf_sc(B, NC, NS)(user_i, item_i, ut_rm, it_rm,
                             w_vec, b_splat)
    return out.reshape(B, 1)


# speedup vs baseline: 15.5333x; 6.0440x over previous
import functools

import jax
import jax.numpy as jnp
from jax import lax
from jax.experimental import pallas as pl
from jax.experimental.pallas import tpu as pltpu
from jax.experimental.pallas import tpu_sc as plsc

K = 16
TW = 128
CH = 16


def _gmf_sc(B, NC, NS):
    NW = NC * NS
    b_per_w = B // NW
    n_chunks = b_per_w // CH
    mesh = plsc.VectorSubcoreMesh(core_axis_name="c", subcore_axis_name="s")

    @functools.partial(
        pl.kernel,
        mesh=mesh,
        out_type=jax.ShapeDtypeStruct((B,), jnp.float32),
        compiler_params=pltpu.CompilerParams(
            needs_layout_passes=False, use_tc_tiling_on_sc=True),
        scratch_types=[
            pltpu.VMEM((b_per_w,), jnp.int32),
            pltpu.VMEM((b_per_w,), jnp.int32),
            pltpu.VMEM((CH, K, TW), jnp.float32),
            pltpu.VMEM((CH, K, TW), jnp.float32),
            pltpu.VMEM((K, K), jnp.float32),
            pltpu.VMEM((K,), jnp.float32),
            pltpu.VMEM((b_per_w,), jnp.float32),
            pltpu.SemaphoreType.DMA,
        ],
    )
    def gmf(user_hbm, item_hbm, utT_hbm, itT_hbm, w_hbm, b_hbm, out_hbm,
            uidxv, iidxv, ubufs, ibufs, wv, bv, outv, sem):
        wid = lax.axis_index("s") * NC + lax.axis_index("c")
        base = wid * b_per_w

        pltpu.sync_copy(w_hbm, wv)
        pltpu.sync_copy(b_hbm, bv)
        pltpu.sync_copy(user_hbm.at[pl.ds(base, b_per_w)], uidxv)
        pltpu.sync_copy(item_hbm.at[pl.ds(base, b_per_w)], iidxv)

        bias = bv[...]
        lane = lax.iota(jnp.int32, K)
        masks = [lane == j for j in range(K)]
        izero = jnp.zeros((K,), jnp.int32)
        wsp = [wv[k] for k in range(K)]

        def chunk_body(c, carry):
            uv = uidxv[pl.ds(c * CH, CH)]
            iv = iidxv[pl.ds(c * CH, CH)]
            uis = []
            iis = []
            for j in range(CH):
                ui = jnp.sum(jnp.where(masks[j], uv, izero))
                ii = jnp.sum(jnp.where(masks[j], iv, izero))
                uis.append(ui)
                iis.append(ii)
                ub = pl.multiple_of((ui // TW) * TW, TW)
                ib = pl.multiple_of((ii // TW) * TW, TW)
                pltpu.async_copy(
                    utT_hbm.at[:, pl.ds(ub, TW)], ubufs.at[j], sem)
                pltpu.async_copy(
                    itT_hbm.at[:, pl.ds(ib, TW)], ibufs.at[j], sem)
            for j in range(CH):
                pltpu.make_async_copy(
                    utT_hbm.at[:, pl.ds(0, TW)], ubufs.at[j], sem).wait()
                pltpu.make_async_copy(
                    itT_hbm.at[:, pl.ds(0, TW)], ibufs.at[j], sem).wait()

            outacc = bias
            for j in range(CH):
                cu = lax.rem(uis[j], TW)
                ci = lax.rem(iis[j], TW)
                cu_w = (cu // K) * K
                ci_w = (ci // K) * K
                cu_l = jnp.full((K,), lax.rem(cu, K), jnp.int32)
                ci_l = jnp.full((K,), lax.rem(ci, K), jnp.int32)
                acc = jnp.zeros((K,), jnp.float32)
                for k in range(K):
                    u_w = ubufs[j, k, pl.ds(cu_w, K)]
                    v_w = ibufs[j, k, pl.ds(ci_w, K)]
                    us = u_w[cu_l]
                    vs = v_w[ci_l]
                    acc = acc + us * vs * wsp[k]
                outacc = jnp.where(masks[j], acc + bias, outacc)
            outv[pl.ds(c * CH, CH)] = 1.0 / (1.0 + jnp.exp(-outacc))
            return carry

        lax.fori_loop(0, n_chunks, chunk_body, 0)
        pltpu.sync_copy(outv, out_hbm.at[pl.ds(base, b_per_w)])

    return gmf


def kernel(user, item, user_table, item_table, W, b):
    B = user.shape[0]
    info = plsc.get_sparse_core_info()
    NC, NS = info.num_cores, info.num_subcores

    user_i = user.astype(jnp.int32)
    item_i = item.astype(jnp.int32)
    utT = user_table.T
    itT = item_table.T
    w_splat = jnp.broadcast_to(W.reshape(K, 1), (K, K))
    b_splat = jnp.broadcast_to(b.reshape(1), (K,))

    out = _gmf_sc(B, NC, NS)(user_i, item_i, utT, itT, w_splat, b_splat)
    return out.reshape(B, 1)


# speedup vs baseline: 19.7969x; 1.2745x over previous
import functools

import jax
import jax.numpy as jnp
from jax import lax
from jax.experimental import pallas as pl
from jax.experimental.pallas import tpu as pltpu
from jax.experimental.pallas import tpu_sc as plsc

K = 16
TW = 128
CH = 8


def _gmf_sc(B, NC, NS):
    NW = NC * NS
    b_per_w = B // NW
    n_chunks = b_per_w // CH
    mesh = plsc.VectorSubcoreMesh(core_axis_name="c", subcore_axis_name="s")

    @functools.partial(
        pl.kernel,
        mesh=mesh,
        out_type=jax.ShapeDtypeStruct((B,), jnp.float32),
        compiler_params=pltpu.CompilerParams(
            needs_layout_passes=False, use_tc_tiling_on_sc=True),
        scratch_types=[
            pltpu.VMEM((b_per_w,), jnp.int32),
            pltpu.VMEM((b_per_w,), jnp.int32),
            pltpu.VMEM((2, CH, K, TW), jnp.float32),
            pltpu.VMEM((2, CH, K, TW), jnp.float32),
            pltpu.VMEM((K, K), jnp.float32),
            pltpu.VMEM((K,), jnp.float32),
            pltpu.VMEM((b_per_w,), jnp.float32),
            pltpu.SemaphoreType.DMA,
        ],
    )
    def gmf(user_hbm, item_hbm, utT_hbm, itT_hbm, w_hbm, b_hbm, out_hbm,
            uidxv, iidxv, ubufs, ibufs, wv, bv, outv, sem):
        wid = lax.axis_index("s") * NC + lax.axis_index("c")
        base = wid * b_per_w

        pltpu.sync_copy(w_hbm, wv)
        pltpu.sync_copy(b_hbm, bv)
        pltpu.sync_copy(user_hbm.at[pl.ds(base, b_per_w)], uidxv)
        pltpu.sync_copy(item_hbm.at[pl.ds(base, b_per_w)], iidxv)

        bias = bv[...]
        lane = lax.iota(jnp.int32, K)
        masks = [lane == j for j in range(K)]
        izero = jnp.zeros((K,), jnp.int32)
        wsp = [wv[k] for k in range(K)]

        def issue(c, slot, par):
            uv = uidxv[pl.ds((c - par) * CH, K)]
            iv = iidxv[pl.ds((c - par) * CH, K)]
            for j in range(CH):
                ui = jnp.sum(jnp.where(masks[par * CH + j], uv, izero))
                ii = jnp.sum(jnp.where(masks[par * CH + j], iv, izero))
                ub = pl.multiple_of((ui // TW) * TW, TW)
                ib = pl.multiple_of((ii // TW) * TW, TW)
                pltpu.async_copy(
                    utT_hbm.at[:, pl.ds(ub, TW)], ubufs.at[slot, j], sem)
                pltpu.async_copy(
                    itT_hbm.at[:, pl.ds(ib, TW)], ibufs.at[slot, j], sem)

        def drain():
            for j in range(CH):
                pltpu.make_async_copy(
                    utT_hbm.at[:, pl.ds(0, TW)], ubufs.at[0, j], sem).wait()
                pltpu.make_async_copy(
                    itT_hbm.at[:, pl.ds(0, TW)], ibufs.at[0, j], sem).wait()

        def compute(c, slot, outacc, lane_off):
            par = lane_off // CH
            uv = uidxv[pl.ds((c - par) * CH, K)]
            iv = iidxv[pl.ds((c - par) * CH, K)]
            for j in range(CH):
                ui = jnp.sum(jnp.where(masks[lane_off + j], uv, izero))
                ii = jnp.sum(jnp.where(masks[lane_off + j], iv, izero))
                cu = lax.rem(ui, TW)
                ci = lax.rem(ii, TW)
                cu_w = (cu // K) * K
                ci_w = (ci // K) * K
                cu_l = jnp.full((K,), lax.rem(cu, K), jnp.int32)
                ci_l = jnp.full((K,), lax.rem(ci, K), jnp.int32)
                acc = jnp.zeros((K,), jnp.float32)
                for k in range(K):
                    u_w = ubufs[slot, j, k, pl.ds(cu_w, K)]
                    v_w = ibufs[slot, j, k, pl.ds(ci_w, K)]
                    us = u_w[cu_l]
                    vs = v_w[ci_l]
                    acc = acc + us * vs * wsp[k]
                outacc = jnp.where(masks[lane_off + j], acc + bias, outacc)
            return outacc

        issue(0, 0, 0)

        def body2(t, carry):
            c = t * 2
            issue(c + 1, 1, 1)
            drain()
            oa = compute(c, 0, bias, 0)

            @pl.when(t < n_chunks // 2 - 1)
            def _():
                issue(c + 2, 0, 0)

            drain()
            oa = compute(c + 1, 1, oa, CH)
            outv[pl.ds(c * CH, 2 * CH)] = 1.0 / (1.0 + jnp.exp(-oa))
            return carry

        lax.fori_loop(0, n_chunks // 2, body2, 0)
        pltpu.sync_copy(outv, out_hbm.at[pl.ds(base, b_per_w)])

    return gmf


def kernel(user, item, user_table, item_table, W, b):
    B = user.shape[0]
    info = plsc.get_sparse_core_info()
    NC, NS = info.num_cores, info.num_subcores

    user_i = user.astype(jnp.int32)
    item_i = item.astype(jnp.int32)
    utT = user_table.T
    itT = item_table.T
    w_splat = jnp.broadcast_to(W.reshape(K, 1), (K, K))
    b_splat = jnp.broadcast_to(b.reshape(1), (K,))

    out = _gmf_sc(B, NC, NS)(user_i, item_i, utT, itT, w_splat, b_splat)
    return out.reshape(B, 1)


# speedup vs baseline: 20.2901x; 1.0249x over previous
import functools

import jax
import jax.numpy as jnp
from jax import lax
from jax.experimental import pallas as pl
from jax.experimental.pallas import tpu as pltpu
from jax.experimental.pallas import tpu_sc as plsc

K = 16
TW = 128
CH = 8


def _gmf_sc(B, NC, NS):
    NW = NC * NS
    b_per_w = B // NW
    n_chunks = b_per_w // CH
    mesh = plsc.VectorSubcoreMesh(core_axis_name="c", subcore_axis_name="s")

    @functools.partial(
        pl.kernel,
        mesh=mesh,
        out_type=jax.ShapeDtypeStruct((B,), jnp.float32),
        compiler_params=pltpu.CompilerParams(
            needs_layout_passes=False, use_tc_tiling_on_sc=True),
        scratch_types=[
            pltpu.VMEM((b_per_w,), jnp.int32),
            pltpu.VMEM((b_per_w,), jnp.int32),
            pltpu.VMEM((2, CH, K, TW), jnp.float32),
            pltpu.VMEM((2, CH, K, TW), jnp.float32),
            pltpu.VMEM((K,), jnp.float32),
            pltpu.VMEM((K,), jnp.float32),
            pltpu.VMEM((b_per_w,), jnp.float32),
            pltpu.SemaphoreType.DMA,
        ],
    )
    def gmf(user_hbm, item_hbm, utT_hbm, itT_hbm, w_hbm, b_hbm, out_hbm,
            uidxv, iidxv, ubufs, ibufs, wv, bv, outv, sem):
        wid = lax.axis_index("s") * NC + lax.axis_index("c")
        base = wid * b_per_w

        pltpu.sync_copy(w_hbm, wv)
        pltpu.sync_copy(b_hbm, bv)
        pltpu.sync_copy(user_hbm.at[pl.ds(base, b_per_w)], uidxv)
        pltpu.sync_copy(item_hbm.at[pl.ds(base, b_per_w)], iidxv)

        bias = bv[...]
        lane = lax.iota(jnp.int32, K)
        masks = [lane == j for j in range(K)]
        izero = jnp.zeros((K,), jnp.int32)
        wvec = wv[...]

        def issue(c, slot, par):
            uv = uidxv[pl.ds((c - par) * CH, K)]
            iv = iidxv[pl.ds((c - par) * CH, K)]
            for j in range(CH):
                ui = jnp.sum(jnp.where(masks[par * CH + j], uv, izero))
                ii = jnp.sum(jnp.where(masks[par * CH + j], iv, izero))
                ub = pl.multiple_of((ui // TW) * TW, TW)
                ib = pl.multiple_of((ii // TW) * TW, TW)
                pltpu.async_copy(
                    utT_hbm.at[:, pl.ds(ub, TW)], ubufs.at[slot, j], sem)
                pltpu.async_copy(
                    itT_hbm.at[:, pl.ds(ib, TW)], ibufs.at[slot, j], sem)

        def drain():
            for j in range(CH):
                pltpu.make_async_copy(
                    utT_hbm.at[:, pl.ds(0, TW)], ubufs.at[0, j], sem).wait()
                pltpu.make_async_copy(
                    itT_hbm.at[:, pl.ds(0, TW)], ibufs.at[0, j], sem).wait()

        def compute(c, slot, outacc, lane_off):
            par = lane_off // CH
            uv = uidxv[pl.ds((c - par) * CH, K)]
            iv = iidxv[pl.ds((c - par) * CH, K)]
            slotv = jnp.full((K,), slot, jnp.int32)
            for j in range(CH):
                ui = jnp.sum(jnp.where(masks[lane_off + j], uv, izero))
                ii = jnp.sum(jnp.where(masks[lane_off + j], iv, izero))
                cu_s = jnp.full((K,), lax.rem(ui, TW), jnp.int32)
                ci_s = jnp.full((K,), lax.rem(ii, TW), jnp.int32)
                jv = jnp.full((K,), j, jnp.int32)
                u_col = plsc.load_gather(ubufs, [slotv, jv, lane, cu_s])
                v_col = plsc.load_gather(ibufs, [slotv, jv, lane, ci_s])
                s = jnp.sum(u_col * v_col * wvec)
                outacc = jnp.where(masks[lane_off + j], s, outacc)
            return outacc

        issue(0, 0, 0)

        def body2(t, carry):
            c = t * 2
            issue(c + 1, 1, 1)
            drain()
            oa = compute(c, 0, bias, 0)

            @pl.when(t < n_chunks // 2 - 1)
            def _():
                issue(c + 2, 0, 0)

            drain()
            oa = compute(c + 1, 1, oa, CH)
            outv[pl.ds(c * CH, 2 * CH)] = (
                1.0 / (1.0 + jnp.exp(-(oa + bias))))
            return carry

        lax.fori_loop(0, n_chunks // 2, body2, 0)
        pltpu.sync_copy(outv, out_hbm.at[pl.ds(base, b_per_w)])

    return gmf


def kernel(user, item, user_table, item_table, W, b):
    B = user.shape[0]
    info = plsc.get_sparse_core_info()
    NC, NS = info.num_cores, info.num_subcores

    user_i = user.astype(jnp.int32)
    item_i = item.astype(jnp.int32)
    utT = user_table.T
    itT = item_table.T
    w_vec = W.reshape(K)
    b_splat = jnp.broadcast_to(b.reshape(1), (K,))

    out = _gmf_sc(B, NC, NS)(user_i, item_i, utT, itT, w_vec, b_splat)
    return out.reshape(B, 1)
